# baseline (speedup 1.0000x reference)
import jax
import jax.numpy as jnp
from jax.experimental import pallas as pl


D_MODEL = 768
N_ROWS = 9


def _body(x_ref, p_ref, emb_ref, out_ref):
    x = x_ref[...]
    p = p_ref[...]
    emb = emb_ref[...]
    acc = x
    for i in range(N_ROWS):
        acc = acc + p[:, i][:, None] * emb[i][None, :]
    out_ref[...] = acc


def kernel(x, phase_one_hot, emb_table):
    seq, batch, d = x.shape
    n = emb_table.shape[0]
    tokens = seq * batch
    x2 = x.reshape(tokens, d)
    p2 = phase_one_hot.reshape(tokens, n)

    blk = 2048
    grid = (tokens // blk,)
    out = pl.pallas_call(
        _body,
        grid=grid,
        in_specs=[
            pl.BlockSpec((blk, d), lambda i: (i, 0)),
            pl.BlockSpec((blk, n), lambda i: (i, 0)),
            pl.BlockSpec((n, d), lambda i: (0, 0)),
        ],
        out_specs=pl.BlockSpec((blk, d), lambda i: (i, 0)),
        out_shape=jax.ShapeDtypeStruct((tokens, d), x.dtype),
    )(x2, p2, emb_table)
    return out.reshape(seq, batch, d)


# speedup vs baseline: 1.0529x; 1.0529x over previous
import jax
import jax.numpy as jnp
from jax.experimental import pallas as pl


D_MODEL = 768
N_ROWS = 9


def _body(x_ref, p_ref, emb_ref, out_ref):
    s = jnp.dot(p_ref[...], emb_ref[...], preferred_element_type=jnp.float32)
    out_ref[...] = x_ref[...] + s


def kernel(x, phase_one_hot, emb_table):
    seq, batch, d = x.shape
    n = emb_table.shape[0]
    tokens = seq * batch
    x2 = x.reshape(tokens, d)
    p2 = phase_one_hot.reshape(tokens, n)

    blk = 2048
    grid = (tokens // blk,)
    out = pl.pallas_call(
        _body,
        grid=grid,
        in_specs=[
            pl.BlockSpec((blk, d), lambda i: (i, 0)),
            pl.BlockSpec((blk, n), lambda i: (i, 0)),
            pl.BlockSpec((n, d), lambda i: (0, 0)),
        ],
        out_specs=pl.BlockSpec((blk, d), lambda i: (i, 0)),
        out_shape=jax.ShapeDtypeStruct((tokens, d), x.dtype),
    )(x2, p2, emb_table)
    return out.reshape(seq, batch, d)
